# Initial kernel scaffold; baseline (speedup 1.0000x reference)
import jax
import jax.numpy as jnp
from jax.experimental import pallas as pl


def kernel(node_embeddings, edge_embeddings, edge_index_list, ln_g, ln_b, phi_w1, phi_b1, phi_w2, phi_b2, theta_w, theta_b):
    raise NotImplementedError("write your pallas kernel here")


# speedup vs baseline: 2.7275x; 2.7275x over previous
import functools

import jax
import jax.numpy as jnp
from jax import lax
from jax.experimental import pallas as pl
from jax.experimental.pallas import tpu as pltpu
from jax.experimental.pallas import tpu_sc as plsc

NC = 2
NS = 16
CH = 80


def _ln_body(x_ref, g_ref, b_ref, o_ref):
    x = x_ref[...]
    mu = jnp.mean(x, axis=-1, keepdims=True)
    xc = x - mu
    var = jnp.mean(xc * xc, axis=-1, keepdims=True)
    o_ref[...] = xc * lax.rsqrt(var + 1e-5) * g_ref[...] + b_ref[...]


def _mlp_body(e_ref, c_ref, n_ref, w1_ref, b1_ref, w2_ref, b2_ref, msg_ref):
    n = n_ref[...]
    s = e_ref[...] + c_ref[...] + n
    s = s * jax.nn.sigmoid(s)
    h = jnp.dot(s, w1_ref[...], preferred_element_type=jnp.float32) + b1_ref[...]
    h = h * jax.nn.sigmoid(h)
    t = jnp.dot(h, w2_ref[...], preferred_element_type=jnp.float32) + b2_ref[...]
    msg_ref[...] = n * t


def _out_body(x_ref, a0_ref, a1_ref, wt_ref, bt_ref, o_ref):
    t = x_ref[...] + a0_ref[...] + a1_ref[...]
    t = t * jax.nn.sigmoid(t)
    o_ref[...] = jnp.dot(t, wt_ref[...], preferred_element_type=jnp.float32) + bt_ref[...]


def _sc_mesh():
    return plsc.VectorSubcoreMesh(
        core_axis_name="c", subcore_axis_name="s", num_cores=NC, num_subcores=NS)


@functools.lru_cache(maxsize=None)
def _make_gather(N, D, E):
    NW = NC * NS
    EW = E // NW
    n_chunks = EW // CH

    @functools.partial(
        pl.kernel,
        out_type=(jax.ShapeDtypeStruct((E, D), jnp.float32),
                  jax.ShapeDtypeStruct((E, D), jnp.float32)),
        mesh=_sc_mesh(),
        scratch_types=[
            pltpu.VMEM((CH,), jnp.int32),
            pltpu.VMEM((CH,), jnp.int32),
            pltpu.VMEM((CH, D), jnp.float32),
            pltpu.VMEM((CH, D), jnp.float32),
            pltpu.SemaphoreType.DMA,
            pltpu.SemaphoreType.DMA,
        ])
    def gather_k(x_hbm, cidx_hbm, nidx_hbm, cout_hbm, nout_hbm,
                 cidx_v, nidx_v, crow_v, nrow_v, sem1, sem2):
        wid = lax.axis_index("s") * NC + lax.axis_index("c")
        base0 = wid * EW

        def body(i, carry):
            base = base0 + i * CH
            pltpu.sync_copy(cidx_hbm.at[pl.ds(base, CH)], cidx_v)
            pltpu.sync_copy(nidx_hbm.at[pl.ds(base, CH)], nidx_v)
            cp1 = pltpu.async_copy(x_hbm.at[cidx_v], crow_v, sem1)
            cp2 = pltpu.async_copy(x_hbm.at[nidx_v], nrow_v, sem2)
            cp1.wait()
            cp2.wait()
            pltpu.sync_copy(crow_v, cout_hbm.at[pl.ds(base, CH), :])
            pltpu.sync_copy(nrow_v, nout_hbm.at[pl.ds(base, CH), :])
            return carry

        lax.fori_loop(0, n_chunks, body, 0)

    return gather_k


@functools.lru_cache(maxsize=None)
def _make_scatter(N, D, E):
    NW = NC * NS
    EW = E // NW
    n_chunks = EW // CH
    NR = ((N // NS + 7) // 8) * 8
    NR_LAST = N - NR * (NS - 1)
    assert NR_LAST > 0 and NR_LAST % 8 == 0

    @functools.partial(
        pl.kernel,
        out_type=jax.ShapeDtypeStruct((NC, N, D), jnp.float32),
        mesh=_sc_mesh(),
        scratch_types=[
            pltpu.VMEM((CH,), jnp.int32),
            pltpu.VMEM((CH, D), jnp.float32),
            pltpu.VMEM_SHARED((N, D), jnp.float32),
        ])
    def scatter_k(msg_hbm, cidx_hbm, zeros_hbm, agg_hbm, cidx_v, msg_v, agg_sh):
        c = lax.axis_index("c")
        s = lax.axis_index("s")
        wid = s * NC + c
        @pl.when(s < NS - 1)
        def _():
            pltpu.sync_copy(zeros_hbm.at[pl.ds(s * NR, NR), :],
                            agg_sh.at[pl.ds(s * NR, NR), :])

        @pl.when(s == NS - 1)
        def _():
            pltpu.sync_copy(zeros_hbm.at[pl.ds((NS - 1) * NR, NR_LAST), :],
                            agg_sh.at[pl.ds((NS - 1) * NR, NR_LAST), :])

        plsc.subcore_barrier()

        def body(i, carry):
            base = wid * EW + i * CH
            pltpu.sync_copy(cidx_hbm.at[pl.ds(base, CH)], cidx_v)
            pltpu.sync_copy(msg_hbm.at[pl.ds(base, CH), :], msg_v)
            pltpu.sync_copy(msg_v, agg_sh.at[cidx_v], add=True)
            return carry

        lax.fori_loop(0, n_chunks, body, 0)
        plsc.subcore_barrier()

        @pl.when(s < NS - 1)
        def _():
            pltpu.sync_copy(agg_sh.at[pl.ds(s * NR, NR), :],
                            agg_hbm.at[c, pl.ds(s * NR, NR), :])

        @pl.when(s == NS - 1)
        def _():
            pltpu.sync_copy(agg_sh.at[pl.ds((NS - 1) * NR, NR_LAST), :],
                            agg_hbm.at[c, pl.ds((NS - 1) * NR, NR_LAST), :])

    return scatter_k


def kernel(node_embeddings, edge_embeddings, edge_index_list, ln_g, ln_b,
           phi_w1, phi_b1, phi_w2, phi_b2, theta_w, theta_b):
    N, D = node_embeddings.shape
    E = edge_index_list.shape[1]
    H = phi_w1.shape[0]
    assert E % (NC * NS * CH) == 0 and N % NS == 0

    BN = 1000
    assert N % BN == 0
    x = pl.pallas_call(
        _ln_body,
        grid=(N // BN,),
        in_specs=[
            pl.BlockSpec((BN, D), lambda i: (i, 0)),
            pl.BlockSpec((1, D), lambda i: (0, 0)),
            pl.BlockSpec((1, D), lambda i: (0, 0)),
        ],
        out_specs=pl.BlockSpec((BN, D), lambda i: (i, 0)),
        out_shape=jax.ShapeDtypeStruct((N, D), jnp.float32),
    )(node_embeddings, ln_g.reshape(1, D), ln_b.reshape(1, D))

    cidx = edge_index_list[0]
    nidx = edge_index_list[1]
    crows, nrows = _make_gather(N, D, E)(x, cidx, nidx)

    BE = 1280
    assert E % BE == 0
    w1t = phi_w1.T
    w2t = phi_w2.T
    msg = pl.pallas_call(
        _mlp_body,
        grid=(E // BE,),
        in_specs=[
            pl.BlockSpec((BE, D), lambda i: (i, 0)),
            pl.BlockSpec((BE, D), lambda i: (i, 0)),
            pl.BlockSpec((BE, D), lambda i: (i, 0)),
            pl.BlockSpec((D, H), lambda i: (0, 0)),
            pl.BlockSpec((1, H), lambda i: (0, 0)),
            pl.BlockSpec((H, D), lambda i: (0, 0)),
            pl.BlockSpec((1, D), lambda i: (0, 0)),
        ],
        out_specs=pl.BlockSpec((BE, D), lambda i: (i, 0)),
        out_shape=jax.ShapeDtypeStruct((E, D), jnp.float32),
        compiler_params=pltpu.CompilerParams(
            dimension_semantics=("arbitrary",)),
    )(edge_embeddings, crows, nrows, w1t, phi_b1.reshape(1, H),
      w2t, phi_b2.reshape(1, D))

    zeros = jnp.zeros((N, D), jnp.float32)
    aggp = _make_scatter(N, D, E)(msg, cidx, zeros)

    wt = theta_w.T
    out = pl.pallas_call(
        _out_body,
        grid=(N // BN,),
        in_specs=[
            pl.BlockSpec((BN, D), lambda i: (i, 0)),
            pl.BlockSpec((BN, D), lambda i: (i, 0)),
            pl.BlockSpec((BN, D), lambda i: (i, 0)),
            pl.BlockSpec((D, D), lambda i: (0, 0)),
            pl.BlockSpec((1, D), lambda i: (0, 0)),
        ],
        out_specs=pl.BlockSpec((BN, D), lambda i: (i, 0)),
        out_shape=jax.ShapeDtypeStruct((N, D), jnp.float32),
    )(x, aggp[0], aggp[1], wt, theta_b.reshape(1, D))
    return out


# speedup vs baseline: 4.0864x; 1.4982x over previous
import functools

import jax
import jax.numpy as jnp
from jax import lax
from jax.experimental import pallas as pl
from jax.experimental.pallas import tpu as pltpu
from jax.experimental.pallas import tpu_sc as plsc

NC = 2
NS = 16
CH = 80


def _ln_body(x_ref, g_ref, b_ref, o_ref):
    x = x_ref[...]
    mu = jnp.mean(x, axis=-1, keepdims=True)
    xc = x - mu
    var = jnp.mean(xc * xc, axis=-1, keepdims=True)
    o_ref[...] = xc * lax.rsqrt(var + 1e-5) * g_ref[...] + b_ref[...]


def _mlp_body(e_ref, c_ref, n_ref, w1_ref, b1_ref, w2_ref, b2_ref, msg_ref):
    n = n_ref[...]
    s = e_ref[...] + c_ref[...] + n
    s = s * jax.nn.sigmoid(s)
    h = jnp.dot(s, w1_ref[...], preferred_element_type=jnp.float32) + b1_ref[...]
    h = h * jax.nn.sigmoid(h)
    t = jnp.dot(h, w2_ref[...], preferred_element_type=jnp.float32) + b2_ref[...]
    msg_ref[...] = n * t


def _out_body(x_ref, a0_ref, a1_ref, wt_ref, bt_ref, o_ref):
    t = x_ref[...] + a0_ref[...] + a1_ref[...]
    t = t * jax.nn.sigmoid(t)
    o_ref[...] = jnp.dot(t, wt_ref[...], preferred_element_type=jnp.float32) + bt_ref[...]


def _sc_mesh():
    return plsc.VectorSubcoreMesh(
        core_axis_name="c", subcore_axis_name="s", num_cores=NC, num_subcores=NS)


NBUF = 4


@functools.lru_cache(maxsize=None)
def _make_gather(N, D, E):
    NW = NC * NS
    EW = E // NW
    n_chunks = EW // CH
    n_outer = (n_chunks + NBUF - 1) // NBUF
    assert n_chunks >= NBUF

    scratch = (
        [pltpu.VMEM((EW,), jnp.int32)] * 2
        + [pltpu.VMEM((CH, D), jnp.float32)] * (2 * NBUF)
        + [pltpu.SemaphoreType.DMA] * (4 * NBUF)
    )

    @functools.partial(
        pl.kernel,
        out_type=(jax.ShapeDtypeStruct((E, D), jnp.float32),
                  jax.ShapeDtypeStruct((E, D), jnp.float32)),
        mesh=_sc_mesh(),
        scratch_types=scratch)
    def gather_k(x_hbm, cidx_hbm, nidx_hbm, cout_hbm, nout_hbm, *scr):
        cidx_all, nidx_all = scr[0], scr[1]
        crow = scr[2:2 + NBUF]
        nrow = scr[2 + NBUF:2 + 2 * NBUF]
        gc = scr[2 + 2 * NBUF:2 + 3 * NBUF]
        gn = scr[2 + 3 * NBUF:2 + 4 * NBUF]
        wc = scr[2 + 4 * NBUF:2 + 5 * NBUF]
        wn = scr[2 + 5 * NBUF:2 + 6 * NBUF]
        wid = lax.axis_index("s") * NC + lax.axis_index("c")
        e0 = wid * EW
        pltpu.sync_copy(cidx_hbm.at[pl.ds(e0, EW)], cidx_all)
        pltpu.sync_copy(nidx_hbm.at[pl.ds(e0, EW)], nidx_all)

        def g_start(i, b):
            pltpu.async_copy(
                x_hbm.at[cidx_all.at[pl.ds(i * CH, CH)]], crow[b], gc[b])
            pltpu.async_copy(
                x_hbm.at[nidx_all.at[pl.ds(i * CH, CH)]], nrow[b], gn[b])

        def g_wait(b):
            pltpu.make_async_copy(x_hbm.at[pl.ds(0, CH), :], crow[b], gc[b]).wait()
            pltpu.make_async_copy(x_hbm.at[pl.ds(0, CH), :], nrow[b], gn[b]).wait()

        def w_start(i, b):
            pltpu.async_copy(crow[b], cout_hbm.at[pl.ds(e0 + i * CH, CH), :], wc[b])
            pltpu.async_copy(nrow[b], nout_hbm.at[pl.ds(e0 + i * CH, CH), :], wn[b])

        def w_wait(b):
            pltpu.make_async_copy(crow[b], cout_hbm.at[pl.ds(0, CH), :], wc[b]).wait()
            pltpu.make_async_copy(nrow[b], nout_hbm.at[pl.ds(0, CH), :], wn[b]).wait()

        for b in range(NBUF):
            g_start(b, b)

        def outer(j, carry):
            for b in range(NBUF):
                i = j * NBUF + b

                @pl.when(i < n_chunks)
                def _():
                    g_wait(b)
                    w_start(i, b)
                    w_wait(b)

                    @pl.when(i + NBUF < n_chunks)
                    def _():
                        g_start(i + NBUF, b)
            return carry

        lax.fori_loop(0, n_outer, outer, 0)

    return gather_k


@functools.lru_cache(maxsize=None)
def _make_scatter(N, D, E):
    NW = NC * NS
    EW = E // NW
    n_chunks = EW // CH
    NR = ((N // NS + 7) // 8) * 8
    NR_LAST = N - NR * (NS - 1)
    assert NR_LAST > 0 and NR_LAST % 8 == 0

    n_outer = (n_chunks + NBUF - 1) // NBUF
    assert n_chunks >= NBUF
    scratch = (
        [pltpu.VMEM((CH,), jnp.int32)] * NBUF
        + [pltpu.VMEM((CH, D), jnp.float32)] * NBUF
        + [pltpu.VMEM_SHARED((N, D), jnp.float32)]
        + [pltpu.SemaphoreType.DMA] * (3 * NBUF)
    )

    @functools.partial(
        pl.kernel,
        out_type=jax.ShapeDtypeStruct((NC, N, D), jnp.float32),
        mesh=_sc_mesh(),
        scratch_types=scratch)
    def scatter_k(msg_hbm, cidx_hbm, zeros_hbm, agg_hbm, *scr):
        cidx_v = scr[0:NBUF]
        msg_v = scr[NBUF:2 * NBUF]
        agg_sh = scr[2 * NBUF]
        ic = scr[2 * NBUF + 1:2 * NBUF + 1 + NBUF]
        im = scr[2 * NBUF + 1 + NBUF:2 * NBUF + 1 + 2 * NBUF]
        ss = scr[2 * NBUF + 1 + 2 * NBUF:2 * NBUF + 1 + 3 * NBUF]
        c = lax.axis_index("c")
        s = lax.axis_index("s")
        wid = s * NC + c
        e0 = wid * EW
        @pl.when(s < NS - 1)
        def _():
            pltpu.sync_copy(zeros_hbm.at[pl.ds(s * NR, NR), :],
                            agg_sh.at[pl.ds(s * NR, NR), :])

        @pl.when(s == NS - 1)
        def _():
            pltpu.sync_copy(zeros_hbm.at[pl.ds((NS - 1) * NR, NR_LAST), :],
                            agg_sh.at[pl.ds((NS - 1) * NR, NR_LAST), :])

        plsc.subcore_barrier()

        def l_start(i, b):
            pltpu.async_copy(cidx_hbm.at[pl.ds(e0 + i * CH, CH)], cidx_v[b], ic[b])
            pltpu.async_copy(msg_hbm.at[pl.ds(e0 + i * CH, CH), :], msg_v[b], im[b])

        def l_wait(b):
            pltpu.make_async_copy(cidx_hbm.at[pl.ds(0, CH)], cidx_v[b], ic[b]).wait()
            pltpu.make_async_copy(msg_hbm.at[pl.ds(0, CH), :], msg_v[b], im[b]).wait()

        def s_start(b):
            pltpu.async_copy(msg_v[b], agg_sh.at[cidx_v[b]], ss[b], add=True)

        def s_wait(b):
            pltpu.make_async_copy(msg_hbm.at[pl.ds(0, CH), :], msg_v[b], ss[b]).wait()

        for b in range(NBUF):
            l_start(b, b)

        def outer(j, carry):
            for b in range(NBUF):
                i = j * NBUF + b

                @pl.when(i < n_chunks)
                def _():
                    l_wait(b)
                    s_start(b)
                    s_wait(b)

                    @pl.when(i + NBUF < n_chunks)
                    def _():
                        l_start(i + NBUF, b)
            return carry

        lax.fori_loop(0, n_outer, outer, 0)
        plsc.subcore_barrier()

        @pl.when(s < NS - 1)
        def _():
            pltpu.sync_copy(agg_sh.at[pl.ds(s * NR, NR), :],
                            agg_hbm.at[c, pl.ds(s * NR, NR), :])

        @pl.when(s == NS - 1)
        def _():
            pltpu.sync_copy(agg_sh.at[pl.ds((NS - 1) * NR, NR_LAST), :],
                            agg_hbm.at[c, pl.ds((NS - 1) * NR, NR_LAST), :])

    return scatter_k


def kernel(node_embeddings, edge_embeddings, edge_index_list, ln_g, ln_b,
           phi_w1, phi_b1, phi_w2, phi_b2, theta_w, theta_b):
    N, D = node_embeddings.shape
    E = edge_index_list.shape[1]
    H = phi_w1.shape[0]
    assert E % (NC * NS * CH) == 0 and N % NS == 0

    BN = 1000
    assert N % BN == 0
    x = pl.pallas_call(
        _ln_body,
        grid=(N // BN,),
        in_specs=[
            pl.BlockSpec((BN, D), lambda i: (i, 0)),
            pl.BlockSpec((1, D), lambda i: (0, 0)),
            pl.BlockSpec((1, D), lambda i: (0, 0)),
        ],
        out_specs=pl.BlockSpec((BN, D), lambda i: (i, 0)),
        out_shape=jax.ShapeDtypeStruct((N, D), jnp.float32),
    )(node_embeddings, ln_g.reshape(1, D), ln_b.reshape(1, D))

    cidx = edge_index_list[0]
    nidx = edge_index_list[1]
    crows, nrows = _make_gather(N, D, E)(x, cidx, nidx)

    BE = 1280
    assert E % BE == 0
    w1t = phi_w1.T
    w2t = phi_w2.T
    msg = pl.pallas_call(
        _mlp_body,
        grid=(E // BE,),
        in_specs=[
            pl.BlockSpec((BE, D), lambda i: (i, 0)),
            pl.BlockSpec((BE, D), lambda i: (i, 0)),
            pl.BlockSpec((BE, D), lambda i: (i, 0)),
            pl.BlockSpec((D, H), lambda i: (0, 0)),
            pl.BlockSpec((1, H), lambda i: (0, 0)),
            pl.BlockSpec((H, D), lambda i: (0, 0)),
            pl.BlockSpec((1, D), lambda i: (0, 0)),
        ],
        out_specs=pl.BlockSpec((BE, D), lambda i: (i, 0)),
        out_shape=jax.ShapeDtypeStruct((E, D), jnp.float32),
        compiler_params=pltpu.CompilerParams(
            dimension_semantics=("arbitrary",)),
    )(edge_embeddings, crows, nrows, w1t, phi_b1.reshape(1, H),
      w2t, phi_b2.reshape(1, D))

    zeros = jnp.zeros((N, D), jnp.float32)
    aggp = _make_scatter(N, D, E)(msg, cidx, zeros)

    wt = theta_w.T
    out = pl.pallas_call(
        _out_body,
        grid=(N // BN,),
        in_specs=[
            pl.BlockSpec((BN, D), lambda i: (i, 0)),
            pl.BlockSpec((BN, D), lambda i: (i, 0)),
            pl.BlockSpec((BN, D), lambda i: (i, 0)),
            pl.BlockSpec((D, D), lambda i: (0, 0)),
            pl.BlockSpec((1, D), lambda i: (0, 0)),
        ],
        out_specs=pl.BlockSpec((BN, D), lambda i: (i, 0)),
        out_shape=jax.ShapeDtypeStruct((N, D), jnp.float32),
    )(x, aggp[0], aggp[1], wt, theta_b.reshape(1, D))
    return out
